# baseline (speedup 1.0000x reference)
import functools
import jax
import jax.numpy as jnp
from jax import lax
from jax.experimental import pallas as pl
from jax.experimental.pallas import tpu as pltpu
from jax.experimental.pallas import tpu_sc as plsc

B = 1024
V = 1000
D = 64
VP = 1024
L = 16
NC, NS = 2, 16
NW = NC * NS
BPW = B // NW
NG = 4
GR = BPW // NG

_sc_mesh = plsc.VectorSubcoreMesh(core_axis_name="c", subcore_axis_name="s")


def _tc_scores_body(cidx_ref, ec_ref, eo_ref, es_ref):
    col = lax.broadcasted_iota(jnp.int32, (B, VP), 1)
    onehot = jnp.where(col == cidx_ref[...], 1.0, 0.0)
    c = lax.dot_general(onehot, ec_ref[...], (((1,), (0,)), ((), ())),
                        preferred_element_type=jnp.float32)
    s = lax.dot_general(c, eo_ref[...], (((1,), (1,)), ((), ())),
                        preferred_element_type=jnp.float32)
    es_ref[...] = jnp.where(col < V, jnp.exp(s), 0.0)


_tc_scores = pl.pallas_call(
    _tc_scores_body,
    out_shape=jax.ShapeDtypeStruct((B, VP), jnp.float32),
)


@functools.partial(
    pl.kernel,
    out_type=(jax.ShapeDtypeStruct((B, L), jnp.float32),
              jax.ShapeDtypeStruct((B,), jnp.float32)),
    mesh=_sc_mesh,
    scratch_types=[
        pltpu.VMEM((BPW, VP), jnp.float32),
        pltpu.VMEM((BPW, V), jnp.int32),
        pltpu.VMEM((BPW,), jnp.int32),
        pltpu.VMEM((BPW, L), jnp.float32),
        pltpu.VMEM((BPW,), jnp.float32),
        [pltpu.SemaphoreType.DMA] * NG,
    ],
    compiler_params=pltpu.CompilerParams(needs_layout_passes=False,
                                         use_tc_tiling_on_sc=True),
)
def _sc_gather_sum(es_hbm, idx_hbm, oidx_hbm, out_hbm, top_hbm,
                   es_v, idx_v, oidx_v, acc_v, top_v, gsems):
    wid = lax.axis_index("s") * NC + lax.axis_index("c")
    base = wid * BPW
    pltpu.sync_copy(oidx_hbm.at[pl.ds(base, BPW)], oidx_v)
    copies = []
    for g in range(NG):
        copies.append([
            pltpu.async_copy(es_hbm.at[pl.ds(base + g * GR, GR)],
                             es_v.at[pl.ds(g * GR, GR)], gsems[g]),
            pltpu.async_copy(idx_hbm.at[pl.ds(base + g * GR, GR)],
                             idx_v.at[pl.ds(g * GR, GR)], gsems[g]),
        ])

    nfull = V // L
    ntail = V - nfull * L
    lane = lax.iota(jnp.int32, L)

    def row(r, carry):
        rvec = jnp.full((L,), r, jnp.int32)
        acc = jnp.zeros((L,), jnp.float32)
        for k in range(nfull):
            iv = plsc.load_gather(idx_v, [rvec, lane + k * L])
            acc = acc + plsc.load_gather(es_v, [rvec, iv])
        cv = jnp.minimum(lane + nfull * L, V - 1)
        iv = plsc.load_gather(idx_v, [rvec, cv])
        g = plsc.load_gather(es_v, [rvec, iv])
        acc = acc + jnp.where(lane < ntail, g, 0.0)
        plsc.store_scatter(acc_v, [rvec, lane], acc)
        return carry

    for g in range(NG):
        for c in copies[g]:
            c.wait()
        lax.fori_loop(g * GR, (g + 1) * GR, row, 0)

    for c16 in range(BPW // L):
        rv = lane + c16 * L
        ov = plsc.load_gather(oidx_v, [rv])
        plsc.store_scatter(top_v, [rv], plsc.load_gather(es_v, [rv, ov]))

    pltpu.sync_copy(acc_v, out_hbm.at[pl.ds(base, BPW)])
    pltpu.sync_copy(top_v, top_hbm.at[pl.ds(base, BPW)])


def _tc_loss_body(top_ref, part_ref, out_ref):
    lsum = jnp.sum(part_ref[...], axis=1)
    val = jnp.mean(jnp.log(top_ref[...])) - jnp.mean(jnp.log(lsum))
    out_ref[...] = (-val).reshape(1, 1)


_tc_loss = pl.pallas_call(
    _tc_loss_body,
    out_shape=jax.ShapeDtypeStruct((1, 1), jnp.float32),
)


def kernel(center, outside, all_vocabs, emb_center, emb_outside):
    cidx = center if center.dtype == jnp.int32 else center.astype(jnp.int32)
    oidx = outside.reshape(B)
    if oidx.dtype != jnp.int32:
        oidx = oidx.astype(jnp.int32)
    av = all_vocabs if all_vocabs.dtype == jnp.int32 else (
        all_vocabs.astype(jnp.int32))
    ec_pad = jnp.pad(emb_center, ((0, VP - V), (0, 0)))
    eo_pad = jnp.pad(emb_outside, ((0, VP - V), (0, 0)))

    es = _tc_scores(cidx, ec_pad, eo_pad)
    part, top = _sc_gather_sum(es, av, oidx)
    loss = _tc_loss(top.reshape(8, 128), part)
    return loss.reshape(())


# speedup vs baseline: 1.0199x; 1.0199x over previous
import functools
import jax
import jax.numpy as jnp
from jax import lax
from jax.experimental import pallas as pl
from jax.experimental.pallas import tpu as pltpu
from jax.experimental.pallas import tpu_sc as plsc

B = 1024
V = 1000
D = 64
VP = 1024
L = 16
NC, NS = 2, 16
NW = NC * NS
BPW = B // NW
NG = 4
GR = BPW // NG

_sc_mesh = plsc.VectorSubcoreMesh(core_axis_name="c", subcore_axis_name="s")


def _tc_scores_body(cidx_ref, ec_ref, eo_ref, es_ref):
    col = lax.broadcasted_iota(jnp.int32, (B, VP), 1)
    onehot = jnp.where(col == cidx_ref[...], 1.0, 0.0)
    c = lax.dot_general(onehot, ec_ref[...], (((1,), (0,)), ((), ())),
                        preferred_element_type=jnp.float32)
    s = lax.dot_general(c, eo_ref[...], (((1,), (1,)), ((), ())),
                        preferred_element_type=jnp.float32)
    es_ref[...] = jnp.where(col < V, jnp.exp(s), 0.0)


_tc_scores = pl.pallas_call(
    _tc_scores_body,
    out_shape=jax.ShapeDtypeStruct((B, VP), jnp.float32),
)


@functools.partial(
    pl.kernel,
    out_type=(jax.ShapeDtypeStruct((B, L), jnp.float32),
              jax.ShapeDtypeStruct((B,), jnp.float32)),
    mesh=_sc_mesh,
    scratch_types=[
        pltpu.VMEM((BPW, VP), jnp.float32),
        pltpu.VMEM((BPW, V), jnp.int32),
        pltpu.VMEM((BPW,), jnp.int32),
        pltpu.VMEM((BPW, L), jnp.float32),
        pltpu.VMEM((BPW,), jnp.float32),
        [pltpu.SemaphoreType.DMA] * NG,
    ],
    compiler_params=pltpu.CompilerParams(needs_layout_passes=False),
)
def _sc_gather_sum(es_hbm, idx_hbm, oidx_hbm, out_hbm, top_hbm,
                   es_v, idx_v, oidx_v, acc_v, top_v, gsems):
    wid = lax.axis_index("s") * NC + lax.axis_index("c")
    base = wid * BPW
    pltpu.sync_copy(oidx_hbm.at[pl.ds(base, BPW)], oidx_v)
    pltpu.sync_copy(es_hbm.at[pl.ds(base, BPW)], es_v)
    pltpu.sync_copy(idx_hbm.at[pl.ds(base, BPW)], idx_v)

    nfull = V // L
    ntail = V - nfull * L
    lane = lax.iota(jnp.int32, L)

    def one_row_acc(rvec):
        acc = jnp.zeros((L,), jnp.float32)
        for k in range(nfull):
            iv = plsc.load_gather(idx_v, [rvec, lane + k * L])
            acc = acc + plsc.load_gather(es_v, [rvec, iv])
        cv = jnp.minimum(lane + nfull * L, V - 1)
        iv = plsc.load_gather(idx_v, [rvec, cv])
        g = plsc.load_gather(es_v, [rvec, iv])
        return acc + jnp.where(lane < ntail, g, 0.0)

    def row_pair(i, carry):
        r0 = i * 2
        rv0 = jnp.full((L,), r0, jnp.int32)
        rv1 = jnp.full((L,), r0 + 1, jnp.int32)
        plsc.store_scatter(acc_v, [rv0, lane], one_row_acc(rv0))
        plsc.store_scatter(acc_v, [rv1, lane], one_row_acc(rv1))
        return carry

    lax.fori_loop(0, BPW // 2, row_pair, 0)

    for c16 in range(BPW // L):
        rv = lane + c16 * L
        ov = plsc.load_gather(oidx_v, [rv])
        plsc.store_scatter(top_v, [rv], plsc.load_gather(es_v, [rv, ov]))

    pltpu.sync_copy(acc_v, out_hbm.at[pl.ds(base, BPW)])
    pltpu.sync_copy(top_v, top_hbm.at[pl.ds(base, BPW)])


def _tc_loss_body(top_ref, part_ref, out_ref):
    lsum = jnp.sum(part_ref[...], axis=1)
    val = jnp.mean(jnp.log(top_ref[...])) - jnp.mean(jnp.log(lsum))
    out_ref[...] = (-val).reshape(1, 1)


_tc_loss = pl.pallas_call(
    _tc_loss_body,
    out_shape=jax.ShapeDtypeStruct((1, 1), jnp.float32),
)


def kernel(center, outside, all_vocabs, emb_center, emb_outside):
    cidx = center if center.dtype == jnp.int32 else center.astype(jnp.int32)
    oidx = outside.reshape(B)
    if oidx.dtype != jnp.int32:
        oidx = oidx.astype(jnp.int32)
    av = all_vocabs if all_vocabs.dtype == jnp.int32 else (
        all_vocabs.astype(jnp.int32))
    ec_pad = jnp.pad(emb_center, ((0, VP - V), (0, 0)))
    eo_pad = jnp.pad(emb_outside, ((0, VP - V), (0, 0)))

    es = _tc_scores(cidx, ec_pad, eo_pad)
    part, top = _sc_gather_sum(es, av, oidx)
    loss = _tc_loss(top.reshape(8, 128), part)
    return loss.reshape(())
